# Initial kernel scaffold; baseline (speedup 1.0000x reference)
import jax
import jax.numpy as jnp
from jax.experimental import pallas as pl


def kernel(action_logps, stop_logps, start_logps, actions):
    raise NotImplementedError("write your pallas kernel here")


# speedup vs baseline: 2.0832x; 2.0832x over previous
import jax
import jax.numpy as jnp
from jax.experimental import pallas as pl
from jax.experimental.pallas import tpu as pltpu

_L = 128


def _fwd_kernel(ap_ref, oh_ref, beta_ref, omb_ref, st_ref, blast_ref,
                out_ref, f_s, p_s, q_s):
    g = pl.program_id(0)
    n = pl.num_programs(0)
    L = ap_ref.shape[0]

    acv = jnp.sum(ap_ref[...] * oh_ref[...], axis=2)
    p_s[...] = st_ref[...] + acv
    q_s[...] = omb_ref[...] + acv

    @pl.when(g == 0)
    def _():
        f_s[...] = p_s[0:1, :]

    def body(t, f):
        beta_t = beta_ref[pl.ds(t, 1), :]
        v = f + beta_t
        m = jnp.max(v, axis=1, keepdims=True)
        s = jnp.log(jnp.sum(jnp.exp(v - m), axis=1, keepdims=True)) + m
        x = s + p_s[pl.ds(t, 1), :]
        y = f + q_s[pl.ds(t, 1), :]
        return jnp.maximum(x, y) + jnp.log1p(jnp.exp(-jnp.abs(x - y)))

    t0 = jnp.where(g == 0, 1, 0)
    f = jax.lax.fori_loop(t0, L, body, f_s[...])
    f_s[...] = f

    @pl.when(g == n - 1)
    def _():
        v = f + blast_ref[...]
        m = jnp.max(v, axis=1, keepdims=True)
        s = jnp.log(jnp.sum(jnp.exp(v - m), axis=1, keepdims=True)) + m
        out_ref[...] = -s


def kernel(action_logps, stop_logps, start_logps, actions):
    T = actions.shape[0]
    B = start_logps.shape[1]
    A = action_logps.shape[2]
    L = _L
    n = T // L

    beta = stop_logps[:, :, 0]
    omb = stop_logps[:, :, 1]
    beta_last = beta[T:T + 1]
    oh = jax.nn.one_hot(actions, A, dtype=jnp.float32).reshape(T, 1, A)

    out = pl.pallas_call(
        _fwd_kernel,
        grid=(n,),
        in_specs=[
            pl.BlockSpec((L, B, A), lambda g: (g, 0, 0)),
            pl.BlockSpec((L, 1, A), lambda g: (g, 0, 0)),
            pl.BlockSpec((L, B), lambda g: (g, 0)),
            pl.BlockSpec((L, B), lambda g: (g, 0)),
            pl.BlockSpec((L, B), lambda g: (g, 0)),
            pl.BlockSpec((1, B), lambda g: (0, 0)),
        ],
        out_specs=pl.BlockSpec((1, 1), lambda g: (0, 0)),
        out_shape=jax.ShapeDtypeStruct((1, 1), jnp.float32),
        scratch_shapes=[
            pltpu.VMEM((1, B), jnp.float32),
            pltpu.VMEM((L, B), jnp.float32),
            pltpu.VMEM((L, B), jnp.float32),
        ],
        compiler_params=pltpu.CompilerParams(
            dimension_semantics=("arbitrary",),
        ),
    )(action_logps, oh, beta, omb, start_logps, beta_last)
    return out[0, 0]


# speedup vs baseline: 3.4539x; 1.6579x over previous
import jax
import jax.numpy as jnp
from jax.experimental import pallas as pl
from jax.experimental.pallas import tpu as pltpu

_L = 128
_RS = 4
_LN2 = 0.6931471805599453


def _step(F, t, eb_s, ep_s, eq_s):
    row = pl.ds(t, 1)
    S = jnp.sum(F * eb_s[row, :], axis=1, keepdims=True)
    return S * ep_s[row, :] + F * eq_s[row, :]


def _rescale(F, ei):
    m = jnp.max(F, axis=1, keepdims=True)
    bits = jax.lax.bitcast_convert_type(m, jnp.int32)
    e = ((bits >> 23) & 0xFF) - 127
    sc = jax.lax.bitcast_convert_type((127 - e) << 23, jnp.float32)
    return F * sc, ei + e


def _fwd_kernel(ap_ref, oh_ref, beta_ref, omb_ref, st_ref, blast_ref,
                out_ref, f_s, ei_s, eb_s, ep_s, eq_s):
    g = pl.program_id(0)
    n = pl.num_programs(0)
    L = ap_ref.shape[0]

    acv = jnp.sum(ap_ref[...] * oh_ref[...], axis=2)
    eb_s[...] = jnp.exp(beta_ref[...])
    ep_s[...] = jnp.exp(st_ref[...] + acv)
    eq_s[...] = jnp.exp(omb_ref[...] + acv)

    @pl.when(g == 0)
    def _():
        F = ep_s[0:1, :]
        for t in range(1, _RS):
            F = _step(F, t, eb_s, ep_s, eq_s)
        F, ei = _rescale(F, jnp.zeros((1, 1), jnp.int32))
        f_s[...] = F
        ei_s[...] = ei

    def macro(k, carry):
        F, ei = carry
        base = k * _RS
        for r in range(_RS):
            F = _step(F, base + r, eb_s, ep_s, eq_s)
        return _rescale(F, ei)

    k0 = jnp.where(g == 0, 1, 0)
    F, ei = jax.lax.fori_loop(k0, L // _RS, macro, (f_s[...], ei_s[...]))
    f_s[...] = F
    ei_s[...] = ei

    @pl.when(g == n - 1)
    def _():
        S = jnp.sum(F * jnp.exp(blast_ref[...]), axis=1, keepdims=True)
        out_ref[...] = -(ei.astype(jnp.float32) * _LN2 + jnp.log(S))


def kernel(action_logps, stop_logps, start_logps, actions):
    T = actions.shape[0]
    B = start_logps.shape[1]
    A = action_logps.shape[2]
    L = _L
    n = T // L

    beta = stop_logps[:, :, 0]
    omb = stop_logps[:, :, 1]
    beta_last = beta[T:T + 1]
    oh = jax.nn.one_hot(actions, A, dtype=jnp.float32).reshape(T, 1, A)

    out = pl.pallas_call(
        _fwd_kernel,
        grid=(n,),
        in_specs=[
            pl.BlockSpec((L, B, A), lambda g: (g, 0, 0)),
            pl.BlockSpec((L, 1, A), lambda g: (g, 0, 0)),
            pl.BlockSpec((L, B), lambda g: (g, 0)),
            pl.BlockSpec((L, B), lambda g: (g, 0)),
            pl.BlockSpec((L, B), lambda g: (g, 0)),
            pl.BlockSpec((1, B), lambda g: (0, 0)),
        ],
        out_specs=pl.BlockSpec((1, 1), lambda g: (0, 0)),
        out_shape=jax.ShapeDtypeStruct((1, 1), jnp.float32),
        scratch_shapes=[
            pltpu.VMEM((1, B), jnp.float32),
            pltpu.VMEM((1, 1), jnp.int32),
            pltpu.VMEM((L, B), jnp.float32),
            pltpu.VMEM((L, B), jnp.float32),
            pltpu.VMEM((L, B), jnp.float32),
        ],
        compiler_params=pltpu.CompilerParams(
            dimension_semantics=("arbitrary",),
        ),
    )(action_logps, oh, beta, omb, start_logps, beta_last)
    return out[0, 0]


# speedup vs baseline: 5.6119x; 1.6248x over previous
import jax
import jax.numpy as jnp
from jax.experimental import pallas as pl
from jax.experimental.pallas import tpu as pltpu

_L = 128
_K = 4
_LN2 = 0.6931471805599453


def _step(F, t, eb_s, ep_s, eq_s):
    row = pl.ds(t, 1)
    S = jnp.sum(F * eb_s[row, :], axis=1, keepdims=True)
    return S * ep_s[row, :] + F * eq_s[row, :]


def _exp_of(x):
    bits = jax.lax.bitcast_convert_type(x, jnp.int32)
    return ((bits >> 23) & 0xFF) - 127


def _fwd_kernel(ap_ref, oh_ref, beta_ref, omb_ref, st_ref, blast_ref,
                out_ref, f_s, ei_s, eb_s, ep_s, eq_s):
    g = pl.program_id(0)
    n = pl.num_programs(0)
    L = ap_ref.shape[0]

    acv = jnp.sum(ap_ref[...] * oh_ref[...], axis=2)
    eb_s[...] = jnp.exp(beta_ref[...])
    ep_s[...] = jnp.exp(st_ref[...] + acv)
    eq_s[...] = jnp.exp(omb_ref[...] + acv)

    @pl.when(g == 0)
    def _():
        F = ep_s[0:1, :]
        for t in range(1, _K):
            F = _step(F, t, eb_s, ep_s, eq_s)
        e = _exp_of(jnp.max(F, axis=1, keepdims=True))
        sc = jax.lax.bitcast_convert_type((127 - e) << 23, jnp.float32)
        f_s[...] = F * sc
        ei_s[...] = e

    def macro(k, carry):
        F, ei = carry
        base = k * _K
        ebm = eb_s[pl.ds(base, _K), :]
        epm = ep_s[pl.ds(base, _K), :]
        eqm = eq_s[pl.ds(base, _K), :]
        eb0, eb1, eb2, eb3 = (ebm[j:j + 1, :] for j in range(4))
        ep0, ep1, ep2, ep3 = (epm[j:j + 1, :] for j in range(4))
        eq0, eq1, eq2, eq3 = (eqm[j:j + 1, :] for j in range(4))

        w1 = eq0
        w2 = w1 * eq1
        w3 = w2 * eq2
        u = jnp.concatenate([eb0, eb1 * w1, eb2 * w2, eb3 * w3], axis=0)
        r02 = ep0 * eq1
        r03 = r02 * eq2
        r13 = ep1 * eq2
        kr = jnp.concatenate([
            ep0 * eb1,
            r02 * eb2, ep1 * eb2,
            r03 * eb3, r13 * eb3, ep2 * eb3,
        ], axis=0)
        kv = jnp.sum(kr, axis=1, keepdims=True)
        v1 = eq3 * eq2
        v0 = v1 * eq1
        rm = jnp.concatenate([ep0 * v0, ep1 * v1, ep2 * eq3, ep3], axis=0)
        q = v0 * eq0

        d = jnp.sum(F * u, axis=1, keepdims=True)
        s0 = d[0:1]
        s1 = d[1:2] + kv[0:1] * s0
        s2 = d[2:3] + kv[1:2] * s0 + kv[2:3] * s1
        s3 = d[3:4] + kv[3:4] * s0 + kv[4:5] * s1 + kv[5:6] * s2
        sv = jnp.concatenate([s0, s1, s2, s3], axis=0)
        F = F * q + jnp.sum(rm * sv, axis=0, keepdims=True)

        e = _exp_of(s3)
        sc = jax.lax.bitcast_convert_type((127 - e) << 23, jnp.float32)
        return F * sc, ei + e

    k0 = jnp.where(g == 0, 1, 0)
    F, ei = jax.lax.fori_loop(k0, L // _K, macro, (f_s[...], ei_s[...]))
    f_s[...] = F
    ei_s[...] = ei

    @pl.when(g == n - 1)
    def _():
        S = jnp.sum(F * jnp.exp(blast_ref[...]), axis=1, keepdims=True)
        out_ref[...] = -(ei.astype(jnp.float32) * _LN2 + jnp.log(S))


def kernel(action_logps, stop_logps, start_logps, actions):
    T = actions.shape[0]
    B = start_logps.shape[1]
    A = action_logps.shape[2]
    L = _L
    n = T // L

    beta = stop_logps[:, :, 0]
    omb = stop_logps[:, :, 1]
    beta_last = beta[T:T + 1]
    oh = jax.nn.one_hot(actions, A, dtype=jnp.float32).reshape(T, 1, A)

    out = pl.pallas_call(
        _fwd_kernel,
        grid=(n,),
        in_specs=[
            pl.BlockSpec((L, B, A), lambda g: (g, 0, 0)),
            pl.BlockSpec((L, 1, A), lambda g: (g, 0, 0)),
            pl.BlockSpec((L, B), lambda g: (g, 0)),
            pl.BlockSpec((L, B), lambda g: (g, 0)),
            pl.BlockSpec((L, B), lambda g: (g, 0)),
            pl.BlockSpec((1, B), lambda g: (0, 0)),
        ],
        out_specs=pl.BlockSpec((1, 1), lambda g: (0, 0)),
        out_shape=jax.ShapeDtypeStruct((1, 1), jnp.float32),
        scratch_shapes=[
            pltpu.VMEM((1, B), jnp.float32),
            pltpu.VMEM((1, 1), jnp.int32),
            pltpu.VMEM((L, B), jnp.float32),
            pltpu.VMEM((L, B), jnp.float32),
            pltpu.VMEM((L, B), jnp.float32),
        ],
        compiler_params=pltpu.CompilerParams(
            dimension_semantics=("arbitrary",),
        ),
    )(action_logps, oh, beta, omb, start_logps, beta_last)
    return out[0, 0]


# speedup vs baseline: 7.0929x; 1.2639x over previous
import jax
import jax.numpy as jnp
from jax.experimental import pallas as pl
from jax.experimental.pallas import tpu as pltpu

_L = 128
_K = 8
_LN2 = 0.6931471805599453


def _step(F, t, eb_s, ep_s, eq_s):
    row = pl.ds(t, 1)
    S = jnp.sum(F * eb_s[row, :], axis=1, keepdims=True)
    return S * ep_s[row, :] + F * eq_s[row, :]


def _exp_of(x):
    bits = jax.lax.bitcast_convert_type(x, jnp.int32)
    return ((bits >> 23) & 0xFF) - 127


def _fwd_kernel(ap_ref, oh_ref, beta_ref, omb_ref, st_ref, blast_ref,
                out_ref, f_s, ei_s, eb_s, ep_s, eq_s):
    g = pl.program_id(0)
    n = pl.num_programs(0)
    L = ap_ref.shape[0]

    acv = jnp.sum(ap_ref[...] * oh_ref[...], axis=2)
    eb_s[...] = jnp.exp(beta_ref[...])
    ep_s[...] = jnp.exp(st_ref[...] + acv)
    eq_s[...] = jnp.exp(omb_ref[...] + acv)

    @pl.when(g == 0)
    def _():
        F = ep_s[0:1, :]
        for t in range(1, _K):
            F = _step(F, t, eb_s, ep_s, eq_s)
        e = _exp_of(jnp.max(F, axis=1, keepdims=True))
        sc = jax.lax.bitcast_convert_type((127 - e) << 23, jnp.float32)
        f_s[...] = F * sc
        ei_s[...] = e

    def macro(k, carry):
        F, ei = carry
        base = k * _K
        ebm = eb_s[pl.ds(base, _K), :]
        epm = ep_s[pl.ds(base, _K), :]
        eqm = eq_s[pl.ds(base, _K), :]
        eb = [ebm[j:j + 1, :] for j in range(_K)]
        ep = [epm[j:j + 1, :] for j in range(_K)]
        eq = [eqm[j:j + 1, :] for j in range(_K)]

        u = [eb[0]]
        w = eq[0]
        for j in range(1, _K):
            u.append(eb[j] * w)
            if j < _K - 1:
                w = w * eq[j]
        kr = []
        for i in range(_K - 1):
            acc = ep[i]
            for j in range(i + 1, _K):
                kr.append(acc * eb[j])
                if j < _K - 1:
                    acc = acc * eq[j]
        kv = jnp.sum(jnp.concatenate(kr, axis=0), axis=1,
                     keepdims=True)
        v = [None] * _K
        v[_K - 1] = ep[_K - 1]
        vacc = eq[_K - 1]
        for j in range(_K - 2, -1, -1):
            v[j] = ep[j] * vacc
            if j > 0:
                vacc = vacc * eq[j]
        rm = jnp.concatenate(v, axis=0)
        q = vacc * eq[0]

        um = jnp.concatenate(u, axis=0)
        d = jnp.sum(F * um, axis=1, keepdims=True)

        def kidx(i, j):
            return i * (_K - 1) - i * (i - 1) // 2 + (j - i - 1)

        s = [d[0:1]]
        for j in range(1, _K):
            acc = d[j:j + 1]
            for i in range(j):
                acc = acc + kv[kidx(i, j):kidx(i, j) + 1] * s[i]
            s.append(acc)
        sv = jnp.concatenate(s, axis=0)
        F = F * q + jnp.sum(rm * sv, axis=0, keepdims=True)

        e = _exp_of(s[_K - 1])
        sc = jax.lax.bitcast_convert_type((127 - e) << 23, jnp.float32)
        return F * sc, ei + e

    k0 = jnp.where(g == 0, 1, 0)
    F, ei = jax.lax.fori_loop(k0, L // _K, macro, (f_s[...], ei_s[...]))
    f_s[...] = F
    ei_s[...] = ei

    @pl.when(g == n - 1)
    def _():
        S = jnp.sum(F * jnp.exp(blast_ref[...]), axis=1, keepdims=True)
        out_ref[...] = -(ei.astype(jnp.float32) * _LN2 + jnp.log(S))


def kernel(action_logps, stop_logps, start_logps, actions):
    T = actions.shape[0]
    B = start_logps.shape[1]
    A = action_logps.shape[2]
    L = _L
    n = T // L

    beta = stop_logps[:, :, 0]
    omb = stop_logps[:, :, 1]
    beta_last = beta[T:T + 1]
    oh = jax.nn.one_hot(actions, A, dtype=jnp.float32).reshape(T, 1, A)

    out = pl.pallas_call(
        _fwd_kernel,
        grid=(n,),
        in_specs=[
            pl.BlockSpec((L, B, A), lambda g: (g, 0, 0)),
            pl.BlockSpec((L, 1, A), lambda g: (g, 0, 0)),
            pl.BlockSpec((L, B), lambda g: (g, 0)),
            pl.BlockSpec((L, B), lambda g: (g, 0)),
            pl.BlockSpec((L, B), lambda g: (g, 0)),
            pl.BlockSpec((1, B), lambda g: (0, 0)),
        ],
        out_specs=pl.BlockSpec((1, 1), lambda g: (0, 0)),
        out_shape=jax.ShapeDtypeStruct((1, 1), jnp.float32),
        scratch_shapes=[
            pltpu.VMEM((1, B), jnp.float32),
            pltpu.VMEM((1, 1), jnp.int32),
            pltpu.VMEM((L, B), jnp.float32),
            pltpu.VMEM((L, B), jnp.float32),
            pltpu.VMEM((L, B), jnp.float32),
        ],
        compiler_params=pltpu.CompilerParams(
            dimension_semantics=("arbitrary",),
        ),
    )(action_logps, oh, beta, omb, start_logps, beta_last)
    return out[0, 0]


# speedup vs baseline: 7.4254x; 1.0469x over previous
import jax
import jax.numpy as jnp
from jax.experimental import pallas as pl
from jax.experimental.pallas import tpu as pltpu

_L = 128
_K = 8
_LN2 = 0.6931471805599453


def _step(F, t, eb_s, ep_s, eq_s):
    row = pl.ds(t, 1)
    S = jnp.sum(F * eb_s[row, :], axis=1, keepdims=True)
    return S * ep_s[row, :] + F * eq_s[row, :]


def _exp_of(x):
    bits = jax.lax.bitcast_convert_type(x, jnp.int32)
    return ((bits >> 23) & 0xFF) - 127


def _fwd_kernel(ap_ref, oh_ref, beta_ref, omb_ref, st_ref, blast_ref,
                out_ref, f_s, ei_s, eb_s, ep_s, eq_s):
    g = pl.program_id(0)
    n = pl.num_programs(0)
    L = ap_ref.shape[0]
    B = ap_ref.shape[1]

    acv = jnp.sum(ap_ref[...] * oh_ref[...], axis=2)
    eb_s[...] = jnp.exp(beta_ref[...])
    ep_s[...] = jnp.exp(st_ref[...] + acv)
    eq_s[...] = jnp.exp(omb_ref[...] + acv)

    @pl.when(g == 0)
    def _():
        F = ep_s[0:1, :]
        for t in range(1, _K):
            F = _step(F, t, eb_s, ep_s, eq_s)
        e = _exp_of(jnp.max(F, axis=1, keepdims=True))
        sc = jax.lax.bitcast_convert_type((127 - e) << 23, jnp.float32)
        f_s[...] = F * sc
        ei_s[...] = e

    def macro(k, carry):
        F, ei = carry
        base = k * _K
        ebm = eb_s[pl.ds(base, _K), :]
        epm = ep_s[pl.ds(base, _K), :]
        eqm = eq_s[pl.ds(base, _K), :]
        one = jnp.ones((1, B), jnp.float32)

        def shift_down(x, k):
            return jnp.concatenate(
                [jnp.broadcast_to(one, (k, B)), x[:_K - k]], axis=0)

        def shift_up(x, k):
            return jnp.concatenate(
                [x[k:], jnp.broadcast_to(one, (k, B))], axis=0)

        w = shift_down(eqm, 1)
        w = w * shift_down(w, 1)
        w = w * shift_down(w, 2)
        w = w * shift_down(w, 4)
        um = ebm * w
        v = shift_up(eqm, 1)
        v = v * shift_up(v, 1)
        v = v * shift_up(v, 2)
        v = v * shift_up(v, 4)
        rm = epm * v
        q = v[0:1] * eqm[0:1]
        kvrows = []
        acc = epm[0:_K - 1]
        for dd in range(1, _K):
            kvrows.append(acc[0:_K - dd] * ebm[dd:])
            if dd < _K - 1:
                acc = acc[0:_K - 1 - dd] * eqm[dd:_K - 1]
        kv = jnp.sum(jnp.concatenate(kvrows, axis=0), axis=1,
                     keepdims=True)

        d = jnp.sum(F * um, axis=1, keepdims=True)

        def kidx(i, j):
            dd = j - i
            return (dd - 1) * _K - (dd - 1) * dd // 2 + i

        s = [d[0:1]]
        for j in range(1, _K):
            acc = d[j:j + 1]
            for i in range(j):
                acc = acc + kv[kidx(i, j):kidx(i, j) + 1] * s[i]
            s.append(acc)
        sv = jnp.concatenate(s, axis=0)
        F = F * q + jnp.sum(rm * sv, axis=0, keepdims=True)

        e = _exp_of(s[_K - 1])
        sc = jax.lax.bitcast_convert_type((127 - e) << 23, jnp.float32)
        return F * sc, ei + e

    k0 = jnp.where(g == 0, 1, 0)
    F, ei = jax.lax.fori_loop(k0, L // _K, macro, (f_s[...], ei_s[...]))
    f_s[...] = F
    ei_s[...] = ei

    @pl.when(g == n - 1)
    def _():
        S = jnp.sum(F * jnp.exp(blast_ref[...]), axis=1, keepdims=True)
        out_ref[...] = -(ei.astype(jnp.float32) * _LN2 + jnp.log(S))


def kernel(action_logps, stop_logps, start_logps, actions):
    T = actions.shape[0]
    B = start_logps.shape[1]
    A = action_logps.shape[2]
    L = _L
    n = T // L

    beta = stop_logps[:, :, 0]
    omb = stop_logps[:, :, 1]
    beta_last = beta[T:T + 1]
    oh = jax.nn.one_hot(actions, A, dtype=jnp.float32).reshape(T, 1, A)

    out = pl.pallas_call(
        _fwd_kernel,
        grid=(n,),
        in_specs=[
            pl.BlockSpec((L, B, A), lambda g: (g, 0, 0)),
            pl.BlockSpec((L, 1, A), lambda g: (g, 0, 0)),
            pl.BlockSpec((L, B), lambda g: (g, 0)),
            pl.BlockSpec((L, B), lambda g: (g, 0)),
            pl.BlockSpec((L, B), lambda g: (g, 0)),
            pl.BlockSpec((1, B), lambda g: (0, 0)),
        ],
        out_specs=pl.BlockSpec((1, 1), lambda g: (0, 0)),
        out_shape=jax.ShapeDtypeStruct((1, 1), jnp.float32),
        scratch_shapes=[
            pltpu.VMEM((1, B), jnp.float32),
            pltpu.VMEM((1, 1), jnp.int32),
            pltpu.VMEM((L, B), jnp.float32),
            pltpu.VMEM((L, B), jnp.float32),
            pltpu.VMEM((L, B), jnp.float32),
        ],
        compiler_params=pltpu.CompilerParams(
            dimension_semantics=("arbitrary",),
        ),
    )(action_logps, oh, beta, omb, start_logps, beta_last)
    return out[0, 0]


# speedup vs baseline: 7.4668x; 1.0056x over previous
import jax
import jax.numpy as jnp
from jax.experimental import pallas as pl
from jax.experimental.pallas import tpu as pltpu

_L = 256
_K = 8
_LN2 = 0.6931471805599453


def _step(F, t, eb_s, ep_s, eq_s):
    row = pl.ds(t, 1)
    S = jnp.sum(F * eb_s[row, :], axis=1, keepdims=True)
    return S * ep_s[row, :] + F * eq_s[row, :]


def _exp_of(x):
    bits = jax.lax.bitcast_convert_type(x, jnp.int32)
    return ((bits >> 23) & 0xFF) - 127


def _fwd_kernel(ap_ref, oh_ref, beta_ref, omb_ref, st_ref, blast_ref,
                out_ref, f_s, ei_s, eb_s, ep_s, eq_s):
    g = pl.program_id(0)
    n = pl.num_programs(0)
    L = ap_ref.shape[0]
    B = ap_ref.shape[1]

    acv = jnp.sum(ap_ref[...] * oh_ref[...], axis=2)
    eb_s[...] = jnp.exp(beta_ref[...])
    ep_s[...] = jnp.exp(st_ref[...] + acv)
    eq_s[...] = jnp.exp(omb_ref[...] + acv)

    @pl.when(g == 0)
    def _():
        F = ep_s[0:1, :]
        for t in range(1, _K):
            F = _step(F, t, eb_s, ep_s, eq_s)
        e = _exp_of(jnp.max(F, axis=1, keepdims=True))
        sc = jax.lax.bitcast_convert_type((127 - e) << 23, jnp.float32)
        f_s[...] = F * sc
        ei_s[...] = e

    def macro(k, carry):
        F, ei = carry
        base = k * _K
        ebm = eb_s[pl.ds(base, _K), :]
        epm = ep_s[pl.ds(base, _K), :]
        eqm = eq_s[pl.ds(base, _K), :]
        one = jnp.ones((1, B), jnp.float32)

        def shift_down(x, k):
            return jnp.concatenate(
                [jnp.broadcast_to(one, (k, B)), x[:_K - k]], axis=0)

        def shift_up(x, k):
            return jnp.concatenate(
                [x[k:], jnp.broadcast_to(one, (k, B))], axis=0)

        w = shift_down(eqm, 1)
        w = w * shift_down(w, 1)
        w = w * shift_down(w, 2)
        w = w * shift_down(w, 4)
        um = ebm * w
        v = shift_up(eqm, 1)
        v = v * shift_up(v, 1)
        v = v * shift_up(v, 2)
        v = v * shift_up(v, 4)
        rm = epm * v
        q = v[0:1] * eqm[0:1]
        kvrows = []
        acc = epm[0:_K - 1]
        for dd in range(1, _K):
            kvrows.append(acc[0:_K - dd] * ebm[dd:])
            if dd < _K - 1:
                acc = acc[0:_K - 1 - dd] * eqm[dd:_K - 1]
        kv = jnp.sum(jnp.concatenate(kvrows, axis=0), axis=1,
                     keepdims=True)

        d = jnp.sum(F * um, axis=1, keepdims=True)

        def kidx(i, j):
            dd = j - i
            return (dd - 1) * _K - (dd - 1) * dd // 2 + i

        s = [d[0:1]]
        for j in range(1, _K):
            acc = d[j:j + 1]
            for i in range(j):
                acc = acc + kv[kidx(i, j):kidx(i, j) + 1] * s[i]
            s.append(acc)
        sv = jnp.concatenate(s, axis=0)
        F = F * q + jnp.sum(rm * sv, axis=0, keepdims=True)

        e = _exp_of(s[_K - 1])
        sc = jax.lax.bitcast_convert_type((127 - e) << 23, jnp.float32)
        return F * sc, ei + e

    k0 = jnp.where(g == 0, 1, 0)
    F, ei = jax.lax.fori_loop(k0, L // _K, macro, (f_s[...], ei_s[...]))
    f_s[...] = F
    ei_s[...] = ei

    @pl.when(g == n - 1)
    def _():
        S = jnp.sum(F * jnp.exp(blast_ref[...]), axis=1, keepdims=True)
        out_ref[...] = -(ei.astype(jnp.float32) * _LN2 + jnp.log(S))


def kernel(action_logps, stop_logps, start_logps, actions):
    T = actions.shape[0]
    B = start_logps.shape[1]
    A = action_logps.shape[2]
    L = _L
    n = T // L

    beta = stop_logps[:, :, 0]
    omb = stop_logps[:, :, 1]
    beta_last = beta[T:T + 1]
    oh = jax.nn.one_hot(actions, A, dtype=jnp.float32).reshape(T, 1, A)

    out = pl.pallas_call(
        _fwd_kernel,
        grid=(n,),
        in_specs=[
            pl.BlockSpec((L, B, A), lambda g: (g, 0, 0)),
            pl.BlockSpec((L, 1, A), lambda g: (g, 0, 0)),
            pl.BlockSpec((L, B), lambda g: (g, 0)),
            pl.BlockSpec((L, B), lambda g: (g, 0)),
            pl.BlockSpec((L, B), lambda g: (g, 0)),
            pl.BlockSpec((1, B), lambda g: (0, 0)),
        ],
        out_specs=pl.BlockSpec((1, 1), lambda g: (0, 0)),
        out_shape=jax.ShapeDtypeStruct((1, 1), jnp.float32),
        scratch_shapes=[
            pltpu.VMEM((1, B), jnp.float32),
            pltpu.VMEM((1, 1), jnp.int32),
            pltpu.VMEM((L, B), jnp.float32),
            pltpu.VMEM((L, B), jnp.float32),
            pltpu.VMEM((L, B), jnp.float32),
        ],
        compiler_params=pltpu.CompilerParams(
            dimension_semantics=("arbitrary",),
        ),
    )(action_logps, oh, beta, omb, start_logps, beta_last)
    return out[0, 0]
